# baseline (speedup 1.0000x reference)
import functools

import jax
import jax.numpy as jnp
from jax import lax
from jax.experimental import pallas as pl
from jax.experimental.pallas import tpu as pltpu
from jax.experimental.pallas import tpu_sc as plsc

_N = 10000
_E = 320000
_D = 128

_NC = 2
_NS = 16
_NW = _NC * _NS
_EPW = _E // _NW
_CH = 128
_NCH = 80
_PAD = _NCH * _CH - _EPW
_ND = 16
_NA = _N + _ND
_RA = 624
_RREM = _N - _NS * _RA
_L = 16

_mesh = plsc.VectorSubcoreMesh(core_axis_name="c", subcore_axis_name="s")


def _sc_deg_body(srcr, dstr, zeros2, ones_h, out,
                 src_v, dst_v, ones_v, degs_sh, degd_sh, sem_s, sem_d):
    c = lax.axis_index("c")
    s = lax.axis_index("s")
    wid = s * _NC + c
    pltpu.sync_copy(srcr.at[wid], src_v)
    pltpu.sync_copy(dstr.at[wid], dst_v)
    pltpu.sync_copy(ones_h, ones_v)

    @pl.when(s == 0)
    def _():
        pltpu.sync_copy(zeros2.at[0], degs_sh)
        pltpu.sync_copy(zeros2.at[1], degd_sh)

    plsc.subcore_barrier()

    def body(j, carry):
        pltpu.async_copy(ones_v, degs_sh.at[src_v.at[j]], sem_s, add=True)
        pltpu.async_copy(ones_v, degd_sh.at[dst_v.at[j]], sem_d, add=True)
        return carry

    lax.fori_loop(0, _NCH, body, 0)

    def drain(j, carry):
        pltpu.make_async_copy(ones_v, degs_sh.at[src_v.at[0]], sem_s).wait()
        pltpu.make_async_copy(ones_v, degd_sh.at[dst_v.at[0]], sem_d).wait()
        return carry

    lax.fori_loop(0, _NCH, drain, 0)
    plsc.subcore_barrier()

    @pl.when(s == 0)
    def _():
        pltpu.sync_copy(degs_sh, out.at[c, 0])
        pltpu.sync_copy(degd_sh, out.at[c, 1])


_sc_deg = functools.partial(
    pl.kernel,
    _sc_deg_body,
    out_type=jax.ShapeDtypeStruct((_NC, 2, _NA), jnp.float32),
    mesh=_mesh,
    scratch_types=[
        pltpu.VMEM((_NCH, _CH), jnp.int32),
        pltpu.VMEM((_NCH, _CH), jnp.int32),
        pltpu.VMEM((_CH,), jnp.float32),
        pltpu.VMEM_SHARED((_NA,), jnp.float32),
        pltpu.VMEM_SHARED((_NA,), jnp.float32),
        pltpu.SemaphoreType.DMA,
        pltpu.SemaphoreType.DMA,
    ],
)()


def _sc_agg_body(h, pk, zeros, out,
                 pk_v, sa, da, sb, db, rows_a, rows_b, agg_sh,
                 sg_a, sg_b, ss_a, ss_b):
    c = lax.axis_index("c")
    s = lax.axis_index("s")
    wid = s * _NC + c
    pltpu.sync_copy(pk.at[wid], pk_v)
    pltpu.sync_copy(zeros.at[pl.ds(s * _RA, _RA)],
                    agg_sh.at[pl.ds(s * _RA, _RA)])

    @pl.when(s == 0)
    def _():
        pltpu.sync_copy(zeros.at[pl.ds(_NS * _RA, _NA - _NS * _RA)],
                        agg_sh.at[pl.ds(_NS * _RA, _NA - _NS * _RA)])

    plsc.subcore_barrier()

    def unpack(j, s_ref, d_ref):
        def ub(k, carry):
            v = pk_v[j, pl.ds(k * _L, _L)]
            s_ref[pl.ds(k * _L, _L)] = v & 0xFFFF
            d_ref[pl.ds(k * _L, _L)] = v >> 16
            return carry
        lax.fori_loop(0, _CH // _L, ub, 0)

    unpack(0, sa, da)
    pltpu.async_copy(h.at[sa], rows_a, sg_a)
    unpack(1, sb, db)
    pltpu.async_copy(h.at[sb], rows_b, sg_b)

    def body(g, carry):
        j0 = 2 * g
        pltpu.make_async_copy(h.at[sa], rows_a, sg_a).wait()
        pltpu.async_copy(rows_a, agg_sh.at[da], ss_a, add=True)
        pltpu.make_async_copy(h.at[sb], rows_b, sg_b).wait()
        pltpu.async_copy(rows_b, agg_sh.at[db], ss_b, add=True)

        @pl.when(j0 + 2 < _NCH)
        def _():
            pltpu.make_async_copy(rows_a, agg_sh.at[da], ss_a).wait()
            unpack(j0 + 2, sa, da)
            pltpu.async_copy(h.at[sa], rows_a, sg_a)

        @pl.when(j0 + 3 < _NCH)
        def _():
            pltpu.make_async_copy(rows_b, agg_sh.at[db], ss_b).wait()
            unpack(j0 + 3, sb, db)
            pltpu.async_copy(h.at[sb], rows_b, sg_b)

        return carry

    lax.fori_loop(0, _NCH // 2, body, 0)
    pltpu.make_async_copy(rows_a, agg_sh.at[da], ss_a).wait()
    pltpu.make_async_copy(rows_b, agg_sh.at[db], ss_b).wait()

    plsc.subcore_barrier()
    pltpu.sync_copy(agg_sh.at[pl.ds(s * _RA, _RA)],
                    out.at[c].at[pl.ds(s * _RA, _RA)])

    @pl.when(s == 0)
    def _():
        pltpu.sync_copy(agg_sh.at[pl.ds(_NS * _RA, _RREM)],
                        out.at[c].at[pl.ds(_NS * _RA, _RREM)])


_sc_agg = functools.partial(
    pl.kernel,
    _sc_agg_body,
    out_type=jax.ShapeDtypeStruct((_NC, _N, _D), jnp.float32),
    mesh=_mesh,
    scratch_types=[
        pltpu.VMEM((_NCH, _CH), jnp.int32),
        pltpu.VMEM((_CH,), jnp.int32),
        pltpu.VMEM((_CH,), jnp.int32),
        pltpu.VMEM((_CH,), jnp.int32),
        pltpu.VMEM((_CH,), jnp.int32),
        pltpu.VMEM((_CH, _D), jnp.float32),
        pltpu.VMEM((_CH, _D), jnp.float32),
        pltpu.VMEM_SHARED((_NA, _D), jnp.float32),
        pltpu.SemaphoreType.DMA,
        pltpu.SemaphoreType.DMA,
        pltpu.SemaphoreType.DMA,
        pltpu.SemaphoreType.DMA,
    ],
)()


def _tc_norm_body(d_ref, o_ref):
    deg = d_ref[0] + d_ref[1]
    o_ref[...] = lax.rsqrt(jnp.maximum(deg, 1.0))


def _tc_norm(deg_parts):
    return pl.pallas_call(
        _tc_norm_body,
        out_shape=jax.ShapeDtypeStruct((2, _NA), jnp.float32),
    )(deg_parts)


def _tc_mm_body(x_ref, ns_ref, w_ref, o_ref):
    o_ref[...] = jnp.dot(x_ref[...] * ns_ref[...], w_ref[...],
                         preferred_element_type=jnp.float32)


def _tc_mm(x, ns, w):
    blk = 1000
    return pl.pallas_call(
        _tc_mm_body,
        grid=(_N // blk,),
        in_specs=[
            pl.BlockSpec((blk, _D), lambda i: (i, 0)),
            pl.BlockSpec((blk, 1), lambda i: (i, 0)),
            pl.BlockSpec((_D, _D), lambda i: (0, 0)),
        ],
        out_specs=pl.BlockSpec((blk, _D), lambda i: (i, 0)),
        out_shape=jax.ShapeDtypeStruct((_N, _D), jnp.float32),
    )(x, ns, w)


def _tc_mid_body(p_ref, nd_ref, b_ref, a_ref, ns_ref, w_ref, o_ref):
    o = (p_ref[0] + p_ref[1]) * nd_ref[...] + b_ref[...]
    o = jnp.maximum(o, 0.0) + a_ref[...] * jnp.minimum(o, 0.0)
    o_ref[...] = jnp.dot(o * ns_ref[...], w_ref[...],
                         preferred_element_type=jnp.float32)


def _tc_mid(p, nd, b, a, ns, w):
    blk = 1000
    return pl.pallas_call(
        _tc_mid_body,
        grid=(_N // blk,),
        in_specs=[
            pl.BlockSpec((_NC, blk, _D), lambda i: (0, i, 0)),
            pl.BlockSpec((blk, 1), lambda i: (i, 0)),
            pl.BlockSpec((1, _D), lambda i: (0, 0)),
            pl.BlockSpec((1, _D), lambda i: (0, 0)),
            pl.BlockSpec((blk, 1), lambda i: (i, 0)),
            pl.BlockSpec((_D, _D), lambda i: (0, 0)),
        ],
        out_specs=pl.BlockSpec((blk, _D), lambda i: (i, 0)),
        out_shape=jax.ShapeDtypeStruct((_N, _D), jnp.float32),
    )(p, nd, b, a, ns, w)


def _tc_out_body(p_ref, nd_ref, b_ref, a_ref, o_ref):
    o = (p_ref[0] + p_ref[1]) * nd_ref[...] + b_ref[...]
    o_ref[...] = jnp.maximum(o, 0.0) + a_ref[...] * jnp.minimum(o, 0.0)


def _tc_out(p, nd, b, a):
    blk = 1000
    return pl.pallas_call(
        _tc_out_body,
        grid=(_N // blk,),
        in_specs=[
            pl.BlockSpec((_NC, blk, _D), lambda i: (0, i, 0)),
            pl.BlockSpec((blk, 1), lambda i: (i, 0)),
            pl.BlockSpec((1, _D), lambda i: (0, 0)),
            pl.BlockSpec((1, _D), lambda i: (0, 0)),
        ],
        out_specs=pl.BlockSpec((blk, _D), lambda i: (i, 0)),
        out_shape=jax.ShapeDtypeStruct((_N, _D), jnp.float32),
    )(p, nd, b, a)


def kernel(features, edge_index, W1, b1, a1, W2, b2, a2):
    src = edge_index[0].reshape(_NW, _EPW)
    dst = edge_index[1].reshape(_NW, _EPW)
    padi = jnp.arange(_NW * _PAD, dtype=jnp.int32).reshape(_NW, _PAD)
    src_a = jnp.concatenate([src, padi % _N], axis=1)
    dst_a = jnp.concatenate([dst, _N + padi % _ND], axis=1)
    packed = (src_a | (dst_a << 16)).reshape(_NW, _NCH, _CH)
    src_d = jnp.concatenate([src, _N + padi % _ND], axis=1).reshape(
        _NW, _NCH, _CH)
    dst_d = jnp.concatenate([dst, _N + padi % _ND], axis=1).reshape(
        _NW, _NCH, _CH)

    zeros_nd = jnp.zeros((_NA, _D), jnp.float32)
    zeros_deg = jnp.zeros((2, _NA), jnp.float32)
    ones_ch = jnp.ones((_CH,), jnp.float32)

    deg_parts = _sc_deg(src_d, dst_d, zeros_deg, ones_ch)
    norms = _tc_norm(deg_parts)
    ns = norms[0, :_N, None]
    nd = norms[1, :_N, None]

    h1 = _tc_mm(features, ns, W1)
    p1 = _sc_agg(h1, packed, zeros_nd)
    h2 = _tc_mid(p1, nd, b1[None], a1[None], ns, W2)
    p2 = _sc_agg(h2, packed, zeros_nd)
    return _tc_out(p2, nd, b2[None], a2[None])


# speedup vs baseline: 1.2437x; 1.2437x over previous
import functools

import jax
import jax.numpy as jnp
from jax import lax
from jax.experimental import pallas as pl
from jax.experimental.pallas import tpu as pltpu
from jax.experimental.pallas import tpu_sc as plsc

_N = 10000
_E = 320000
_D = 128

_NC = 2
_NS = 16
_NW = _NC * _NS
_EPW = _E // _NW
_CH = 128
_NCH = 80
_PAD = _NCH * _CH - _EPW
_ND = 16
_NA = _N + _ND
_RA = 624
_RREM = _N - _NS * _RA
_L = 16

_mesh = plsc.VectorSubcoreMesh(core_axis_name="c", subcore_axis_name="s")


def _sc_deg_body(srcr, dstr, zeros2, ones_h, out,
                 src_v, dst_v, ones_v, degs_sh, degd_sh, sem_s, sem_d):
    c = lax.axis_index("c")
    s = lax.axis_index("s")
    wid = s * _NC + c
    pltpu.sync_copy(srcr.at[wid], src_v)
    pltpu.sync_copy(dstr.at[wid], dst_v)
    pltpu.sync_copy(ones_h, ones_v)

    @pl.when(s == 0)
    def _():
        pltpu.sync_copy(zeros2.at[0], degs_sh)
        pltpu.sync_copy(zeros2.at[1], degd_sh)

    plsc.subcore_barrier()

    def body(j, carry):
        pltpu.async_copy(ones_v, degs_sh.at[src_v.at[j]], sem_s, add=True)
        pltpu.async_copy(ones_v, degd_sh.at[dst_v.at[j]], sem_d, add=True)
        return carry

    lax.fori_loop(0, _NCH, body, 0)

    def drain(j, carry):
        pltpu.make_async_copy(ones_v, degs_sh.at[src_v.at[0]], sem_s).wait()
        pltpu.make_async_copy(ones_v, degd_sh.at[dst_v.at[0]], sem_d).wait()
        return carry

    lax.fori_loop(0, _NCH, drain, 0)
    plsc.subcore_barrier()

    @pl.when(s == 0)
    def _():
        pltpu.sync_copy(degs_sh, out.at[c, 0])
        pltpu.sync_copy(degd_sh, out.at[c, 1])


_sc_deg = functools.partial(
    pl.kernel,
    _sc_deg_body,
    out_type=jax.ShapeDtypeStruct((_NC, 2, _NA), jnp.float32),
    mesh=_mesh,
    scratch_types=[
        pltpu.VMEM((_NCH, _CH), jnp.int32),
        pltpu.VMEM((_NCH, _CH), jnp.int32),
        pltpu.VMEM((_CH,), jnp.float32),
        pltpu.VMEM_SHARED((_NA,), jnp.float32),
        pltpu.VMEM_SHARED((_NA,), jnp.float32),
        pltpu.SemaphoreType.DMA,
        pltpu.SemaphoreType.DMA,
    ],
)()


def _sc_agg_body(h, pk, zeros, out,
                 pk_v, sa, da, sb, db, rows_a, rows_b, agg_sh,
                 sg_a, sg_b, ss_a, ss_b):
    c = lax.axis_index("c")
    s = lax.axis_index("s")
    wid = s * _NC + c
    pltpu.sync_copy(pk.at[wid], pk_v)
    pltpu.sync_copy(zeros.at[pl.ds(s * _RA, _RA)],
                    agg_sh.at[pl.ds(s * _RA, _RA)])

    @pl.when(s == 0)
    def _():
        pltpu.sync_copy(zeros.at[pl.ds(_NS * _RA, _NA - _NS * _RA)],
                        agg_sh.at[pl.ds(_NS * _RA, _NA - _NS * _RA)])

    plsc.subcore_barrier()

    def unpack(j, s_ref, d_ref):
        def ub(k, carry):
            v = pk_v[j, pl.ds(k * _L, _L)]
            s_ref[pl.ds(k * _L, _L)] = v & 0xFFFF
            d_ref[pl.ds(k * _L, _L)] = v >> 16
            return carry
        lax.fori_loop(0, _CH // _L, ub, 0)

    unpack(0, sa, da)
    pltpu.async_copy(h.at[sa], rows_a, sg_a)
    unpack(1, sb, db)
    pltpu.async_copy(h.at[sb], rows_b, sg_b)

    def body(g, carry):
        j0 = 2 * g
        pltpu.make_async_copy(h.at[sa], rows_a, sg_a).wait()
        pltpu.sync_copy(rows_a, agg_sh.at[da], add=True)

        @pl.when(j0 + 2 < _NCH)
        def _():
            unpack(j0 + 2, sa, da)
            pltpu.async_copy(h.at[sa], rows_a, sg_a)

        pltpu.make_async_copy(h.at[sb], rows_b, sg_b).wait()
        pltpu.sync_copy(rows_b, agg_sh.at[db], add=True)

        @pl.when(j0 + 3 < _NCH)
        def _():
            unpack(j0 + 3, sb, db)
            pltpu.async_copy(h.at[sb], rows_b, sg_b)

        return carry

    lax.fori_loop(0, _NCH // 2, body, 0)

    plsc.subcore_barrier()
    pltpu.sync_copy(agg_sh.at[pl.ds(s * _RA, _RA)],
                    out.at[c].at[pl.ds(s * _RA, _RA)])

    @pl.when(s == 0)
    def _():
        pltpu.sync_copy(agg_sh.at[pl.ds(_NS * _RA, _RREM)],
                        out.at[c].at[pl.ds(_NS * _RA, _RREM)])


_sc_agg = functools.partial(
    pl.kernel,
    _sc_agg_body,
    out_type=jax.ShapeDtypeStruct((_NC, _N, _D), jnp.float32),
    mesh=_mesh,
    scratch_types=[
        pltpu.VMEM((_NCH, _CH), jnp.int32),
        pltpu.VMEM((_CH,), jnp.int32),
        pltpu.VMEM((_CH,), jnp.int32),
        pltpu.VMEM((_CH,), jnp.int32),
        pltpu.VMEM((_CH,), jnp.int32),
        pltpu.VMEM((_CH, _D), jnp.float32),
        pltpu.VMEM((_CH, _D), jnp.float32),
        pltpu.VMEM_SHARED((_NA, _D), jnp.float32),
        pltpu.SemaphoreType.DMA,
        pltpu.SemaphoreType.DMA,
        pltpu.SemaphoreType.DMA,
        pltpu.SemaphoreType.DMA,
    ],
)()


def _tc_norm_body(d_ref, o_ref):
    deg = d_ref[0] + d_ref[1]
    o_ref[...] = lax.rsqrt(jnp.maximum(deg, 1.0))


def _tc_norm(deg_parts):
    return pl.pallas_call(
        _tc_norm_body,
        out_shape=jax.ShapeDtypeStruct((2, _NA), jnp.float32),
    )(deg_parts)


def _tc_mm_body(x_ref, ns_ref, w_ref, o_ref):
    o_ref[...] = jnp.dot(x_ref[...] * ns_ref[...], w_ref[...],
                         preferred_element_type=jnp.float32)


def _tc_mm(x, ns, w):
    blk = 1000
    return pl.pallas_call(
        _tc_mm_body,
        grid=(_N // blk,),
        in_specs=[
            pl.BlockSpec((blk, _D), lambda i: (i, 0)),
            pl.BlockSpec((blk, 1), lambda i: (i, 0)),
            pl.BlockSpec((_D, _D), lambda i: (0, 0)),
        ],
        out_specs=pl.BlockSpec((blk, _D), lambda i: (i, 0)),
        out_shape=jax.ShapeDtypeStruct((_N, _D), jnp.float32),
    )(x, ns, w)


def _tc_mid_body(p_ref, nd_ref, b_ref, a_ref, ns_ref, w_ref, o_ref):
    o = (p_ref[0] + p_ref[1]) * nd_ref[...] + b_ref[...]
    o = jnp.maximum(o, 0.0) + a_ref[...] * jnp.minimum(o, 0.0)
    o_ref[...] = jnp.dot(o * ns_ref[...], w_ref[...],
                         preferred_element_type=jnp.float32)


def _tc_mid(p, nd, b, a, ns, w):
    blk = 1000
    return pl.pallas_call(
        _tc_mid_body,
        grid=(_N // blk,),
        in_specs=[
            pl.BlockSpec((_NC, blk, _D), lambda i: (0, i, 0)),
            pl.BlockSpec((blk, 1), lambda i: (i, 0)),
            pl.BlockSpec((1, _D), lambda i: (0, 0)),
            pl.BlockSpec((1, _D), lambda i: (0, 0)),
            pl.BlockSpec((blk, 1), lambda i: (i, 0)),
            pl.BlockSpec((_D, _D), lambda i: (0, 0)),
        ],
        out_specs=pl.BlockSpec((blk, _D), lambda i: (i, 0)),
        out_shape=jax.ShapeDtypeStruct((_N, _D), jnp.float32),
    )(p, nd, b, a, ns, w)


def _tc_out_body(p_ref, nd_ref, b_ref, a_ref, o_ref):
    o = (p_ref[0] + p_ref[1]) * nd_ref[...] + b_ref[...]
    o_ref[...] = jnp.maximum(o, 0.0) + a_ref[...] * jnp.minimum(o, 0.0)


def _tc_out(p, nd, b, a):
    blk = 1000
    return pl.pallas_call(
        _tc_out_body,
        grid=(_N // blk,),
        in_specs=[
            pl.BlockSpec((_NC, blk, _D), lambda i: (0, i, 0)),
            pl.BlockSpec((blk, 1), lambda i: (i, 0)),
            pl.BlockSpec((1, _D), lambda i: (0, 0)),
            pl.BlockSpec((1, _D), lambda i: (0, 0)),
        ],
        out_specs=pl.BlockSpec((blk, _D), lambda i: (i, 0)),
        out_shape=jax.ShapeDtypeStruct((_N, _D), jnp.float32),
    )(p, nd, b, a)


def kernel(features, edge_index, W1, b1, a1, W2, b2, a2):
    src = edge_index[0].reshape(_NW, _EPW)
    dst = edge_index[1].reshape(_NW, _EPW)
    padi = jnp.arange(_NW * _PAD, dtype=jnp.int32).reshape(_NW, _PAD)
    src_a = jnp.concatenate([src, padi % _N], axis=1)
    dst_a = jnp.concatenate([dst, _N + padi % _ND], axis=1)
    packed = (src_a | (dst_a << 16)).reshape(_NW, _NCH, _CH)
    src_d = jnp.concatenate([src, _N + padi % _ND], axis=1).reshape(
        _NW, _NCH, _CH)
    dst_d = jnp.concatenate([dst, _N + padi % _ND], axis=1).reshape(
        _NW, _NCH, _CH)

    zeros_nd = jnp.zeros((_NA, _D), jnp.float32)
    zeros_deg = jnp.zeros((2, _NA), jnp.float32)
    ones_ch = jnp.ones((_CH,), jnp.float32)

    deg_parts = _sc_deg(src_d, dst_d, zeros_deg, ones_ch)
    norms = _tc_norm(deg_parts)
    ns = norms[0, :_N, None]
    nd = norms[1, :_N, None]

    h1 = _tc_mm(features, ns, W1)
    p1 = _sc_agg(h1, packed, zeros_nd)
    h2 = _tc_mid(p1, nd, b1[None], a1[None], ns, W2)
    p2 = _sc_agg(h2, packed, zeros_nd)
    return _tc_out(p2, nd, b2[None], a2[None])


# speedup vs baseline: 1.2610x; 1.0139x over previous
import functools

import jax
import jax.numpy as jnp
from jax import lax
from jax.experimental import pallas as pl
from jax.experimental.pallas import tpu as pltpu
from jax.experimental.pallas import tpu_sc as plsc

_N = 10000
_E = 320000
_D = 128

_NC = 2
_NS = 16
_NW = _NC * _NS
_EPW = _E // _NW
_CH = 128
_NCH = 80
_PAD = _NCH * _CH - _EPW
_ND = 16
_NA = _N + _ND
_RA = 624
_RREM = _N - _NS * _RA
_L = 16

_mesh = plsc.VectorSubcoreMesh(core_axis_name="c", subcore_axis_name="s")


_DCH = 160


def _sc_norm_body(srcd, dstd, zeros1, ones_h, out_s, out_d,
                  idx_v, ones_v, dv, nv, deg_sh, sem):
    c = lax.axis_index("c")
    s = lax.axis_index("s")

    @pl.when(c == 0)
    def _():
        pltpu.sync_copy(srcd.at[s], idx_v)

    @pl.when(c == 1)
    def _():
        pltpu.sync_copy(dstd.at[s], idx_v)

    pltpu.sync_copy(ones_h, ones_v)

    @pl.when(s == 0)
    def _():
        pltpu.sync_copy(zeros1, deg_sh)

    plsc.subcore_barrier()

    def body(j, carry):
        pltpu.async_copy(ones_v, deg_sh.at[idx_v.at[j]], sem, add=True)
        return carry

    lax.fori_loop(0, _DCH, body, 0)

    def drain(j, carry):
        pltpu.make_async_copy(ones_v, deg_sh.at[idx_v.at[0]], sem).wait()
        return carry

    lax.fori_loop(0, _DCH, drain, 0)
    plsc.subcore_barrier()

    def rsqrt16(x):
        xi = lax.bitcast_convert_type(x, jnp.int32)
        y = lax.bitcast_convert_type(0x5F3759DF - (xi >> 1), jnp.float32)
        for _ in range(3):
            y = y * (1.5 - 0.5 * x * y * y)
        return y

    off = s * _RA
    pltpu.sync_copy(deg_sh.at[pl.ds(off, _RA)], dv)

    def nbody(i, carry):
        x = jnp.maximum(dv[pl.ds(i * _L, _L)], 1.0)
        nv[pl.ds(i * _L, _L)] = rsqrt16(x)
        return carry

    lax.fori_loop(0, _RA // _L, nbody, 0)

    @pl.when(c == 0)
    def _():
        pltpu.sync_copy(nv, out_s.at[pl.ds(off, _RA)])

    @pl.when(c == 1)
    def _():
        pltpu.sync_copy(nv, out_d.at[pl.ds(off, _RA)])

    @pl.when(s == 0)
    def _():
        rem = _NA - _NS * _RA
        pltpu.sync_copy(deg_sh.at[pl.ds(_NS * _RA, rem)],
                        dv.at[pl.ds(0, rem)])

        def rbody(i, carry):
            x = jnp.maximum(dv[pl.ds(i * _L, _L)], 1.0)
            nv[pl.ds(i * _L, _L)] = rsqrt16(x)
            return carry

        lax.fori_loop(0, rem // _L, rbody, 0)

        @pl.when(c == 0)
        def _():
            pltpu.sync_copy(nv.at[pl.ds(0, rem)],
                            out_s.at[pl.ds(_NS * _RA, rem)])

        @pl.when(c == 1)
        def _():
            pltpu.sync_copy(nv.at[pl.ds(0, rem)],
                            out_d.at[pl.ds(_NS * _RA, rem)])


_sc_norm = functools.partial(
    pl.kernel,
    _sc_norm_body,
    out_type=[jax.ShapeDtypeStruct((_NA,), jnp.float32),
              jax.ShapeDtypeStruct((_NA,), jnp.float32)],
    mesh=_mesh,
    scratch_types=[
        pltpu.VMEM((_DCH, _CH), jnp.int32),
        pltpu.VMEM((_CH,), jnp.float32),
        pltpu.VMEM((_RA,), jnp.float32),
        pltpu.VMEM((_RA,), jnp.float32),
        pltpu.VMEM_SHARED((_NA,), jnp.float32),
        pltpu.SemaphoreType.DMA,
    ],
)()


def _sc_agg_body(h, pk, zeros, out,
                 pk_v, sa, da, sb, db, rows_a, rows_b, agg_sh,
                 sg_a, sg_b, ss_a, ss_b):
    c = lax.axis_index("c")
    s = lax.axis_index("s")
    wid = s * _NC + c
    pltpu.sync_copy(pk.at[wid], pk_v)
    pltpu.sync_copy(zeros.at[pl.ds(s * _RA, _RA)],
                    agg_sh.at[pl.ds(s * _RA, _RA)])

    @pl.when(s == 0)
    def _():
        pltpu.sync_copy(zeros.at[pl.ds(_NS * _RA, _NA - _NS * _RA)],
                        agg_sh.at[pl.ds(_NS * _RA, _NA - _NS * _RA)])

    plsc.subcore_barrier()

    def unpack(j, s_ref, d_ref):
        def ub(k, carry):
            v = pk_v[j, pl.ds(k * _L, _L)]
            s_ref[pl.ds(k * _L, _L)] = v & 0xFFFF
            d_ref[pl.ds(k * _L, _L)] = v >> 16
            return carry
        lax.fori_loop(0, _CH // _L, ub, 0)

    unpack(0, sa, da)
    pltpu.async_copy(h.at[sa], rows_a, sg_a)
    unpack(1, sb, db)
    pltpu.async_copy(h.at[sb], rows_b, sg_b)

    def body(g, carry):
        j0 = 2 * g
        pltpu.make_async_copy(h.at[sa], rows_a, sg_a).wait()
        pltpu.sync_copy(rows_a, agg_sh.at[da], add=True)

        @pl.when(j0 + 2 < _NCH)
        def _():
            unpack(j0 + 2, sa, da)
            pltpu.async_copy(h.at[sa], rows_a, sg_a)

        pltpu.make_async_copy(h.at[sb], rows_b, sg_b).wait()
        pltpu.sync_copy(rows_b, agg_sh.at[db], add=True)

        @pl.when(j0 + 3 < _NCH)
        def _():
            unpack(j0 + 3, sb, db)
            pltpu.async_copy(h.at[sb], rows_b, sg_b)

        return carry

    lax.fori_loop(0, _NCH // 2, body, 0)

    plsc.subcore_barrier()
    pltpu.sync_copy(agg_sh.at[pl.ds(s * _RA, _RA)],
                    out.at[c].at[pl.ds(s * _RA, _RA)])

    @pl.when(s == 0)
    def _():
        pltpu.sync_copy(agg_sh.at[pl.ds(_NS * _RA, _RREM)],
                        out.at[c].at[pl.ds(_NS * _RA, _RREM)])


_sc_agg = functools.partial(
    pl.kernel,
    _sc_agg_body,
    out_type=jax.ShapeDtypeStruct((_NC, _N, _D), jnp.float32),
    mesh=_mesh,
    scratch_types=[
        pltpu.VMEM((_NCH, _CH), jnp.int32),
        pltpu.VMEM((_CH,), jnp.int32),
        pltpu.VMEM((_CH,), jnp.int32),
        pltpu.VMEM((_CH,), jnp.int32),
        pltpu.VMEM((_CH,), jnp.int32),
        pltpu.VMEM((_CH, _D), jnp.float32),
        pltpu.VMEM((_CH, _D), jnp.float32),
        pltpu.VMEM_SHARED((_NA, _D), jnp.float32),
        pltpu.SemaphoreType.DMA,
        pltpu.SemaphoreType.DMA,
        pltpu.SemaphoreType.DMA,
        pltpu.SemaphoreType.DMA,
    ],
)()


def _tc_mm_body(x_ref, ns_ref, w_ref, o_ref):
    o_ref[...] = jnp.dot(x_ref[...] * ns_ref[...], w_ref[...],
                         preferred_element_type=jnp.float32)


def _tc_mm(x, ns, w):
    blk = 1000
    return pl.pallas_call(
        _tc_mm_body,
        grid=(_N // blk,),
        in_specs=[
            pl.BlockSpec((blk, _D), lambda i: (i, 0)),
            pl.BlockSpec((blk, 1), lambda i: (i, 0)),
            pl.BlockSpec((_D, _D), lambda i: (0, 0)),
        ],
        out_specs=pl.BlockSpec((blk, _D), lambda i: (i, 0)),
        out_shape=jax.ShapeDtypeStruct((_N, _D), jnp.float32),
    )(x, ns, w)


def _tc_mid_body(p_ref, nd_ref, b_ref, a_ref, ns_ref, w_ref, o_ref):
    o = (p_ref[0] + p_ref[1]) * nd_ref[...] + b_ref[...]
    o = jnp.maximum(o, 0.0) + a_ref[...] * jnp.minimum(o, 0.0)
    o_ref[...] = jnp.dot(o * ns_ref[...], w_ref[...],
                         preferred_element_type=jnp.float32)


def _tc_mid(p, nd, b, a, ns, w):
    blk = 1000
    return pl.pallas_call(
        _tc_mid_body,
        grid=(_N // blk,),
        in_specs=[
            pl.BlockSpec((_NC, blk, _D), lambda i: (0, i, 0)),
            pl.BlockSpec((blk, 1), lambda i: (i, 0)),
            pl.BlockSpec((1, _D), lambda i: (0, 0)),
            pl.BlockSpec((1, _D), lambda i: (0, 0)),
            pl.BlockSpec((blk, 1), lambda i: (i, 0)),
            pl.BlockSpec((_D, _D), lambda i: (0, 0)),
        ],
        out_specs=pl.BlockSpec((blk, _D), lambda i: (i, 0)),
        out_shape=jax.ShapeDtypeStruct((_N, _D), jnp.float32),
    )(p, nd, b, a, ns, w)


def _tc_out_body(p_ref, nd_ref, b_ref, a_ref, o_ref):
    o = (p_ref[0] + p_ref[1]) * nd_ref[...] + b_ref[...]
    o_ref[...] = jnp.maximum(o, 0.0) + a_ref[...] * jnp.minimum(o, 0.0)


def _tc_out(p, nd, b, a):
    blk = 1000
    return pl.pallas_call(
        _tc_out_body,
        grid=(_N // blk,),
        in_specs=[
            pl.BlockSpec((_NC, blk, _D), lambda i: (0, i, 0)),
            pl.BlockSpec((blk, 1), lambda i: (i, 0)),
            pl.BlockSpec((1, _D), lambda i: (0, 0)),
            pl.BlockSpec((1, _D), lambda i: (0, 0)),
        ],
        out_specs=pl.BlockSpec((blk, _D), lambda i: (i, 0)),
        out_shape=jax.ShapeDtypeStruct((_N, _D), jnp.float32),
    )(p, nd, b, a)


def kernel(features, edge_index, W1, b1, a1, W2, b2, a2):
    src = edge_index[0].reshape(_NW, _EPW)
    dst = edge_index[1].reshape(_NW, _EPW)
    padi = jnp.arange(_NW * _PAD, dtype=jnp.int32).reshape(_NW, _PAD)
    src_a = jnp.concatenate([src, padi % _N], axis=1)
    dst_a = jnp.concatenate([dst, _N + padi % _ND], axis=1)
    packed = (src_a | (dst_a << 16)).reshape(_NW, _NCH, _CH)
    ept = _E // _NS
    padt = _N + (jnp.arange(_NS * (_DCH * _CH - ept), dtype=jnp.int32)
                 % _ND).reshape(_NS, _DCH * _CH - ept)
    src_d = jnp.concatenate(
        [edge_index[0].reshape(_NS, ept), padt], axis=1).reshape(
            _NS, _DCH, _CH)
    dst_d = jnp.concatenate(
        [edge_index[1].reshape(_NS, ept), padt], axis=1).reshape(
            _NS, _DCH, _CH)

    zeros_nd = jnp.zeros((_NA, _D), jnp.float32)
    zeros_deg = jnp.zeros((_NA,), jnp.float32)
    ones_ch = jnp.ones((_CH,), jnp.float32)

    ns_arr, nd_arr = _sc_norm(src_d, dst_d, zeros_deg, ones_ch)
    ns = ns_arr[:_N, None]
    nd = nd_arr[:_N, None]

    h1 = _tc_mm(features, ns, W1)
    p1 = _sc_agg(h1, packed, zeros_nd)
    h2 = _tc_mid(p1, nd, b1[None], a1[None], ns, W2)
    p2 = _sc_agg(h2, packed, zeros_nd)
    return _tc_out(p2, nd, b2[None], a2[None])


# speedup vs baseline: 1.2721x; 1.0088x over previous
import functools

import jax
import jax.numpy as jnp
from jax import lax
from jax.experimental import pallas as pl
from jax.experimental.pallas import tpu as pltpu
from jax.experimental.pallas import tpu_sc as plsc

_N = 10000
_E = 320000
_D = 128

_NC = 2
_NS = 16
_NW = _NC * _NS
_CH = 128
_NR = _E // _CH
_WCH = 80
_WLAST = _NR - (_NW - 1) * _WCH
_TCH = 160
_TLAST = _NR - (_NS - 1) * _TCH
_RA = 624
_RREM = _N - _NS * _RA
_L = 16

_mesh = plsc.VectorSubcoreMesh(core_axis_name="c", subcore_axis_name="s")


def _sc_norm_body(srcd, dstd, zeros1, ones_h, out_s, out_d,
                  idx_v, ones_v, dv, nv, deg_sh, sem):
    c = lax.axis_index("c")
    s = lax.axis_index("s")
    nch = jnp.where(s == _NS - 1, _TLAST, _TCH)

    @pl.when((c == 0) & (s < _NS - 1))
    def _():
        pltpu.sync_copy(srcd.at[pl.ds(s * _TCH, _TCH)], idx_v)

    @pl.when((c == 0) & (s == _NS - 1))
    def _():
        pltpu.sync_copy(srcd.at[pl.ds((_NS - 1) * _TCH, _TLAST)],
                        idx_v.at[pl.ds(0, _TLAST)])

    @pl.when((c == 1) & (s < _NS - 1))
    def _():
        pltpu.sync_copy(dstd.at[pl.ds(s * _TCH, _TCH)], idx_v)

    @pl.when((c == 1) & (s == _NS - 1))
    def _():
        pltpu.sync_copy(dstd.at[pl.ds((_NS - 1) * _TCH, _TLAST)],
                        idx_v.at[pl.ds(0, _TLAST)])

    pltpu.sync_copy(ones_h, ones_v)

    @pl.when(s == 0)
    def _():
        pltpu.sync_copy(zeros1, deg_sh)

    plsc.subcore_barrier()

    def body(j, carry):
        pltpu.async_copy(ones_v, deg_sh.at[idx_v.at[j]], sem, add=True)
        return carry

    lax.fori_loop(0, nch, body, 0)

    def drain(j, carry):
        pltpu.make_async_copy(ones_v, deg_sh.at[idx_v.at[0]], sem).wait()
        return carry

    lax.fori_loop(0, nch, drain, 0)
    plsc.subcore_barrier()

    def rsqrt16(x):
        xi = lax.bitcast_convert_type(x, jnp.int32)
        y = lax.bitcast_convert_type(0x5F3759DF - (xi >> 1), jnp.float32)
        for _ in range(3):
            y = y * (1.5 - 0.5 * x * y * y)
        return y

    off = s * _RA
    pltpu.sync_copy(deg_sh.at[pl.ds(off, _RA)], dv)

    def nbody(i, carry):
        x = jnp.maximum(dv[pl.ds(i * _L, _L)], 1.0)
        nv[pl.ds(i * _L, _L)] = rsqrt16(x)
        return carry

    lax.fori_loop(0, _RA // _L, nbody, 0)

    @pl.when(c == 0)
    def _():
        pltpu.sync_copy(nv, out_s.at[pl.ds(off, _RA)])

    @pl.when(c == 1)
    def _():
        pltpu.sync_copy(nv, out_d.at[pl.ds(off, _RA)])

    @pl.when(s == 0)
    def _():
        pltpu.sync_copy(deg_sh.at[pl.ds(_NS * _RA, _RREM)],
                        dv.at[pl.ds(0, _RREM)])

        def rbody(i, carry):
            x = jnp.maximum(dv[pl.ds(i * _L, _L)], 1.0)
            nv[pl.ds(i * _L, _L)] = rsqrt16(x)
            return carry

        lax.fori_loop(0, _RREM // _L, rbody, 0)

        @pl.when(c == 0)
        def _():
            pltpu.sync_copy(nv.at[pl.ds(0, _RREM)],
                            out_s.at[pl.ds(_NS * _RA, _RREM)])

        @pl.when(c == 1)
        def _():
            pltpu.sync_copy(nv.at[pl.ds(0, _RREM)],
                            out_d.at[pl.ds(_NS * _RA, _RREM)])


_sc_norm = functools.partial(
    pl.kernel,
    _sc_norm_body,
    out_type=[jax.ShapeDtypeStruct((_N,), jnp.float32),
              jax.ShapeDtypeStruct((_N,), jnp.float32)],
    mesh=_mesh,
    scratch_types=[
        pltpu.VMEM((_TCH, _CH), jnp.int32),
        pltpu.VMEM((_CH,), jnp.float32),
        pltpu.VMEM((_RA,), jnp.float32),
        pltpu.VMEM((_RA,), jnp.float32),
        pltpu.VMEM_SHARED((_N,), jnp.float32),
        pltpu.SemaphoreType.DMA,
    ],
)()


def _sc_agg_body(h, pk, zeros, out,
                 pk_v, sa, da, sb, db, rows_a, rows_b, agg_sh,
                 sg_a, sg_b):
    c = lax.axis_index("c")
    s = lax.axis_index("s")
    wid = s * _NC + c
    nch = jnp.where(wid == _NW - 1, _WLAST, _WCH)

    @pl.when(wid < _NW - 1)
    def _():
        pltpu.sync_copy(pk.at[pl.ds(wid * _WCH, _WCH)], pk_v)

    @pl.when(wid == _NW - 1)
    def _():
        pltpu.sync_copy(pk.at[pl.ds((_NW - 1) * _WCH, _WLAST)],
                        pk_v.at[pl.ds(0, _WLAST)])

    pltpu.sync_copy(zeros.at[pl.ds(s * _RA, _RA)],
                    agg_sh.at[pl.ds(s * _RA, _RA)])

    @pl.when(s == 0)
    def _():
        pltpu.sync_copy(zeros.at[pl.ds(_NS * _RA, _RREM)],
                        agg_sh.at[pl.ds(_NS * _RA, _RREM)])

    plsc.subcore_barrier()

    def unpack(j, s_ref, d_ref):
        def ub(k, carry):
            v = pk_v[j, pl.ds(k * _L, _L)]
            s_ref[pl.ds(k * _L, _L)] = v & 0xFFFF
            d_ref[pl.ds(k * _L, _L)] = v >> 16
            return carry
        lax.fori_loop(0, _CH // _L, ub, 0)

    unpack(0, sa, da)
    pltpu.async_copy(h.at[sa], rows_a, sg_a)
    unpack(1, sb, db)
    pltpu.async_copy(h.at[sb], rows_b, sg_b)

    def body(g, carry):
        j0 = 2 * g
        pltpu.make_async_copy(h.at[sa], rows_a, sg_a).wait()
        pltpu.sync_copy(rows_a, agg_sh.at[da], add=True)

        @pl.when(j0 + 2 < nch)
        def _():
            unpack(j0 + 2, sa, da)
            pltpu.async_copy(h.at[sa], rows_a, sg_a)

        pltpu.make_async_copy(h.at[sb], rows_b, sg_b).wait()
        pltpu.sync_copy(rows_b, agg_sh.at[db], add=True)

        @pl.when(j0 + 3 < nch)
        def _():
            unpack(j0 + 3, sb, db)
            pltpu.async_copy(h.at[sb], rows_b, sg_b)

        return carry

    lax.fori_loop(0, nch // 2, body, 0)
    plsc.subcore_barrier()
    pltpu.sync_copy(agg_sh.at[pl.ds(s * _RA, _RA)],
                    out.at[c].at[pl.ds(s * _RA, _RA)])

    @pl.when(s == 0)
    def _():
        pltpu.sync_copy(agg_sh.at[pl.ds(_NS * _RA, _RREM)],
                        out.at[c].at[pl.ds(_NS * _RA, _RREM)])


_sc_agg = functools.partial(
    pl.kernel,
    _sc_agg_body,
    out_type=jax.ShapeDtypeStruct((_NC, _N, _D), jnp.float32),
    mesh=_mesh,
    scratch_types=[
        pltpu.VMEM((_WCH, _CH), jnp.int32),
        pltpu.VMEM((_CH,), jnp.int32),
        pltpu.VMEM((_CH,), jnp.int32),
        pltpu.VMEM((_CH,), jnp.int32),
        pltpu.VMEM((_CH,), jnp.int32),
        pltpu.VMEM((_CH, _D), jnp.float32),
        pltpu.VMEM((_CH, _D), jnp.float32),
        pltpu.VMEM_SHARED((_N, _D), jnp.float32),
        pltpu.SemaphoreType.DMA,
        pltpu.SemaphoreType.DMA,
    ],
)()


def _tc_mm_body(x_ref, ns_ref, w_ref, o_ref):
    o_ref[...] = jnp.dot(x_ref[...] * ns_ref[...], w_ref[...],
                         preferred_element_type=jnp.float32)


def _tc_mm(x, ns, w):
    blk = 1000
    return pl.pallas_call(
        _tc_mm_body,
        grid=(_N // blk,),
        in_specs=[
            pl.BlockSpec((blk, _D), lambda i: (i, 0)),
            pl.BlockSpec((blk, 1), lambda i: (i, 0)),
            pl.BlockSpec((_D, _D), lambda i: (0, 0)),
        ],
        out_specs=pl.BlockSpec((blk, _D), lambda i: (i, 0)),
        out_shape=jax.ShapeDtypeStruct((_N, _D), jnp.float32),
    )(x, ns, w)


def _tc_mid_body(p_ref, nd_ref, b_ref, a_ref, ns_ref, w_ref, o_ref):
    o = (p_ref[0] + p_ref[1]) * nd_ref[...] + b_ref[...]
    o = jnp.maximum(o, 0.0) + a_ref[...] * jnp.minimum(o, 0.0)
    o_ref[...] = jnp.dot(o * ns_ref[...], w_ref[...],
                         preferred_element_type=jnp.float32)


def _tc_mid(p, nd, b, a, ns, w):
    blk = 1000
    return pl.pallas_call(
        _tc_mid_body,
        grid=(_N // blk,),
        in_specs=[
            pl.BlockSpec((_NC, blk, _D), lambda i: (0, i, 0)),
            pl.BlockSpec((blk, 1), lambda i: (i, 0)),
            pl.BlockSpec((1, _D), lambda i: (0, 0)),
            pl.BlockSpec((1, _D), lambda i: (0, 0)),
            pl.BlockSpec((blk, 1), lambda i: (i, 0)),
            pl.BlockSpec((_D, _D), lambda i: (0, 0)),
        ],
        out_specs=pl.BlockSpec((blk, _D), lambda i: (i, 0)),
        out_shape=jax.ShapeDtypeStruct((_N, _D), jnp.float32),
    )(p, nd, b, a, ns, w)


def _tc_out_body(p_ref, nd_ref, b_ref, a_ref, o_ref):
    o = (p_ref[0] + p_ref[1]) * nd_ref[...] + b_ref[...]
    o_ref[...] = jnp.maximum(o, 0.0) + a_ref[...] * jnp.minimum(o, 0.0)


def _tc_out(p, nd, b, a):
    blk = 1000
    return pl.pallas_call(
        _tc_out_body,
        grid=(_N // blk,),
        in_specs=[
            pl.BlockSpec((_NC, blk, _D), lambda i: (0, i, 0)),
            pl.BlockSpec((blk, 1), lambda i: (i, 0)),
            pl.BlockSpec((1, _D), lambda i: (0, 0)),
            pl.BlockSpec((1, _D), lambda i: (0, 0)),
        ],
        out_specs=pl.BlockSpec((blk, _D), lambda i: (i, 0)),
        out_shape=jax.ShapeDtypeStruct((_N, _D), jnp.float32),
    )(p, nd, b, a)


def kernel(features, edge_index, W1, b1, a1, W2, b2, a2):
    src = edge_index[0]
    dst = edge_index[1]
    packed = (src | (dst << 16)).reshape(_NR, _CH)
    src_d = src.reshape(_NR, _CH)
    dst_d = dst.reshape(_NR, _CH)

    zeros_nd = jnp.zeros((_N, _D), jnp.float32)
    zeros_deg = jnp.zeros((_N,), jnp.float32)
    ones_ch = jnp.ones((_CH,), jnp.float32)

    ns_arr, nd_arr = _sc_norm(src_d, dst_d, zeros_deg, ones_ch)
    ns = ns_arr[:, None]
    nd = nd_arr[:, None]

    h1 = _tc_mm(features, ns, W1)
    p1 = _sc_agg(h1, packed, zeros_nd)
    h2 = _tc_mid(p1, nd, b1[None], a1[None], ns, W2)
    p2 = _sc_agg(h2, packed, zeros_nd)
    return _tc_out(p2, nd, b2[None], a2[None])


# speedup vs baseline: 1.3290x; 1.0447x over previous
import functools

import jax
import jax.numpy as jnp
from jax import lax
from jax.experimental import pallas as pl
from jax.experimental.pallas import tpu as pltpu
from jax.experimental.pallas import tpu_sc as plsc

_N = 10000
_E = 320000
_D = 128

_NC = 2
_NS = 16
_NW = _NC * _NS
_CH = 128
_NR = _E // _CH
_WCH = 80
_WLAST = _NR - (_NW - 1) * _WCH
_TCH = 160
_TLAST = _NR - (_NS - 1) * _TCH
_RA = 624
_RREM = _N - _NS * _RA
_L = 16

_mesh = plsc.VectorSubcoreMesh(core_axis_name="c", subcore_axis_name="s")


def _sc_norm_body(ei3, zeros1, ones_h, out_s, out_d,
                  idx_v, ones_v, dv, nv, deg_sh, sem):
    c = lax.axis_index("c")
    s = lax.axis_index("s")
    nch = jnp.where(s == _NS - 1, _TLAST, _TCH)

    @pl.when(s < _NS - 1)
    def _():
        pltpu.sync_copy(ei3.at[c, pl.ds(s * _TCH, _TCH)], idx_v)

    @pl.when(s == _NS - 1)
    def _():
        pltpu.sync_copy(ei3.at[c, pl.ds((_NS - 1) * _TCH, _TLAST)],
                        idx_v.at[pl.ds(0, _TLAST)])

    pltpu.sync_copy(ones_h, ones_v)

    @pl.when(s == 0)
    def _():
        pltpu.sync_copy(zeros1, deg_sh)

    plsc.subcore_barrier()

    def body(j, carry):
        pltpu.async_copy(ones_v, deg_sh.at[idx_v.at[j]], sem, add=True)
        return carry

    lax.fori_loop(0, nch, body, 0)

    def drain(j, carry):
        pltpu.make_async_copy(ones_v, deg_sh.at[idx_v.at[0]], sem).wait()
        return carry

    lax.fori_loop(0, nch, drain, 0)
    plsc.subcore_barrier()

    def rsqrt16(x):
        xi = lax.bitcast_convert_type(x, jnp.int32)
        y = lax.bitcast_convert_type(0x5F3759DF - (xi >> 1), jnp.float32)
        for _ in range(3):
            y = y * (1.5 - 0.5 * x * y * y)
        return y

    off = s * _RA
    pltpu.sync_copy(deg_sh.at[pl.ds(off, _RA)], dv)

    def nbody(i, carry):
        x = jnp.maximum(dv[pl.ds(i * _L, _L)], 1.0)
        nv[pl.ds(i * _L, _L)] = rsqrt16(x)
        return carry

    lax.fori_loop(0, _RA // _L, nbody, 0)

    @pl.when(c == 0)
    def _():
        pltpu.sync_copy(nv, out_s.at[pl.ds(off, _RA)])

    @pl.when(c == 1)
    def _():
        pltpu.sync_copy(nv, out_d.at[pl.ds(off, _RA)])

    @pl.when(s == 0)
    def _():
        pltpu.sync_copy(deg_sh.at[pl.ds(_NS * _RA, _RREM)],
                        dv.at[pl.ds(0, _RREM)])

        def rbody(i, carry):
            x = jnp.maximum(dv[pl.ds(i * _L, _L)], 1.0)
            nv[pl.ds(i * _L, _L)] = rsqrt16(x)
            return carry

        lax.fori_loop(0, _RREM // _L, rbody, 0)

        @pl.when(c == 0)
        def _():
            pltpu.sync_copy(nv.at[pl.ds(0, _RREM)],
                            out_s.at[pl.ds(_NS * _RA, _RREM)])

        @pl.when(c == 1)
        def _():
            pltpu.sync_copy(nv.at[pl.ds(0, _RREM)],
                            out_d.at[pl.ds(_NS * _RA, _RREM)])


_sc_norm = functools.partial(
    pl.kernel,
    _sc_norm_body,
    out_type=[jax.ShapeDtypeStruct((_N,), jnp.float32),
              jax.ShapeDtypeStruct((_N,), jnp.float32)],
    mesh=_mesh,
    scratch_types=[
        pltpu.VMEM((_TCH, _CH), jnp.int32),
        pltpu.VMEM((_CH,), jnp.float32),
        pltpu.VMEM((_RA,), jnp.float32),
        pltpu.VMEM((_RA,), jnp.float32),
        pltpu.VMEM_SHARED((_N,), jnp.float32),
        pltpu.SemaphoreType.DMA,
    ],
)()


def _sc_agg_body(h, pk, zeros, out,
                 pk_v, sa, da, sb, db, rows_a, rows_b, agg_sh,
                 sg_a, sg_b):
    c = lax.axis_index("c")
    s = lax.axis_index("s")
    wid = s * _NC + c
    nch = jnp.where(wid == _NW - 1, _WLAST, _WCH)

    @pl.when(wid < _NW - 1)
    def _():
        pltpu.sync_copy(pk.at[pl.ds(wid * _WCH, _WCH)], pk_v)

    @pl.when(wid == _NW - 1)
    def _():
        pltpu.sync_copy(pk.at[pl.ds((_NW - 1) * _WCH, _WLAST)],
                        pk_v.at[pl.ds(0, _WLAST)])

    pltpu.sync_copy(zeros.at[pl.ds(s * _RA, _RA)],
                    agg_sh.at[pl.ds(s * _RA, _RA)])

    @pl.when(s == 0)
    def _():
        pltpu.sync_copy(zeros.at[pl.ds(_NS * _RA, _RREM)],
                        agg_sh.at[pl.ds(_NS * _RA, _RREM)])

    plsc.subcore_barrier()

    def unpack(j, s_ref, d_ref):
        def ub(k, carry):
            v = pk_v[j, pl.ds(k * _L, _L)]
            s_ref[pl.ds(k * _L, _L)] = v & 0xFFFF
            d_ref[pl.ds(k * _L, _L)] = v >> 16
            return carry
        lax.fori_loop(0, _CH // _L, ub, 0)

    unpack(0, sa, da)
    pltpu.async_copy(h.at[sa], rows_a, sg_a)
    unpack(1, sb, db)
    pltpu.async_copy(h.at[sb], rows_b, sg_b)

    def body(g, carry):
        j0 = 2 * g
        pltpu.make_async_copy(h.at[sa], rows_a, sg_a).wait()
        pltpu.sync_copy(rows_a, agg_sh.at[da], add=True)

        @pl.when(j0 + 2 < nch)
        def _():
            unpack(j0 + 2, sa, da)
            pltpu.async_copy(h.at[sa], rows_a, sg_a)

        pltpu.make_async_copy(h.at[sb], rows_b, sg_b).wait()
        pltpu.sync_copy(rows_b, agg_sh.at[db], add=True)

        @pl.when(j0 + 3 < nch)
        def _():
            unpack(j0 + 3, sb, db)
            pltpu.async_copy(h.at[sb], rows_b, sg_b)

        return carry

    lax.fori_loop(0, nch // 2, body, 0)
    plsc.subcore_barrier()
    pltpu.sync_copy(agg_sh.at[pl.ds(s * _RA, _RA)],
                    out.at[c].at[pl.ds(s * _RA, _RA)])

    @pl.when(s == 0)
    def _():
        pltpu.sync_copy(agg_sh.at[pl.ds(_NS * _RA, _RREM)],
                        out.at[c].at[pl.ds(_NS * _RA, _RREM)])


_sc_agg = functools.partial(
    pl.kernel,
    _sc_agg_body,
    out_type=jax.ShapeDtypeStruct((_NC, _N, _D), jnp.float32),
    mesh=_mesh,
    scratch_types=[
        pltpu.VMEM((_WCH, _CH), jnp.int32),
        pltpu.VMEM((_CH,), jnp.int32),
        pltpu.VMEM((_CH,), jnp.int32),
        pltpu.VMEM((_CH,), jnp.int32),
        pltpu.VMEM((_CH,), jnp.int32),
        pltpu.VMEM((_CH, _D), jnp.float32),
        pltpu.VMEM((_CH, _D), jnp.float32),
        pltpu.VMEM_SHARED((_N, _D), jnp.float32),
        pltpu.SemaphoreType.DMA,
        pltpu.SemaphoreType.DMA,
    ],
)()


def _tc_mm_body(x_ref, ns_ref, w_ref, o_ref):
    o_ref[...] = jnp.dot(x_ref[...] * ns_ref[...], w_ref[...],
                         preferred_element_type=jnp.float32)


def _tc_mm(x, ns, w):
    blk = 2000
    return pl.pallas_call(
        _tc_mm_body,
        grid=(_N // blk,),
        in_specs=[
            pl.BlockSpec((blk, _D), lambda i: (i, 0)),
            pl.BlockSpec((blk, 1), lambda i: (i, 0)),
            pl.BlockSpec((_D, _D), lambda i: (0, 0)),
        ],
        out_specs=pl.BlockSpec((blk, _D), lambda i: (i, 0)),
        out_shape=jax.ShapeDtypeStruct((_N, _D), jnp.float32),
    )(x, ns, w)


def _tc_mid_body(p_ref, nd_ref, b_ref, a_ref, ns_ref, w_ref, o_ref):
    o = (p_ref[0] + p_ref[1]) * nd_ref[...] + b_ref[...]
    o = jnp.maximum(o, 0.0) + a_ref[...] * jnp.minimum(o, 0.0)
    o_ref[...] = jnp.dot(o * ns_ref[...], w_ref[...],
                         preferred_element_type=jnp.float32)


def _tc_mid(p, nd, b, a, ns, w):
    blk = 2000
    return pl.pallas_call(
        _tc_mid_body,
        grid=(_N // blk,),
        in_specs=[
            pl.BlockSpec((_NC, blk, _D), lambda i: (0, i, 0)),
            pl.BlockSpec((blk, 1), lambda i: (i, 0)),
            pl.BlockSpec((1, _D), lambda i: (0, 0)),
            pl.BlockSpec((1, _D), lambda i: (0, 0)),
            pl.BlockSpec((blk, 1), lambda i: (i, 0)),
            pl.BlockSpec((_D, _D), lambda i: (0, 0)),
        ],
        out_specs=pl.BlockSpec((blk, _D), lambda i: (i, 0)),
        out_shape=jax.ShapeDtypeStruct((_N, _D), jnp.float32),
    )(p, nd, b, a, ns, w)


def _tc_out_body(p_ref, nd_ref, b_ref, a_ref, o_ref):
    o = (p_ref[0] + p_ref[1]) * nd_ref[...] + b_ref[...]
    o_ref[...] = jnp.maximum(o, 0.0) + a_ref[...] * jnp.minimum(o, 0.0)


def _tc_out(p, nd, b, a):
    blk = 2000
    return pl.pallas_call(
        _tc_out_body,
        grid=(_N // blk,),
        in_specs=[
            pl.BlockSpec((_NC, blk, _D), lambda i: (0, i, 0)),
            pl.BlockSpec((blk, 1), lambda i: (i, 0)),
            pl.BlockSpec((1, _D), lambda i: (0, 0)),
            pl.BlockSpec((1, _D), lambda i: (0, 0)),
        ],
        out_specs=pl.BlockSpec((blk, _D), lambda i: (i, 0)),
        out_shape=jax.ShapeDtypeStruct((_N, _D), jnp.float32),
    )(p, nd, b, a)


def kernel(features, edge_index, W1, b1, a1, W2, b2, a2):
    ei3 = edge_index.reshape(2, _NR, _CH)
    packed = ei3[0] | (ei3[1] << 16)

    zeros_nd = jnp.zeros((_N, _D), jnp.float32)
    zeros_deg = jnp.zeros((_N,), jnp.float32)
    ones_ch = jnp.ones((_CH,), jnp.float32)

    ns_arr, nd_arr = _sc_norm(ei3, zeros_deg, ones_ch)
    ns = ns_arr[:, None]
    nd = nd_arr[:, None]

    h1 = _tc_mm(features, ns, W1)
    p1 = _sc_agg(h1, packed, zeros_nd)
    h2 = _tc_mid(p1, nd, b1[None], a1[None], ns, W2)
    p2 = _sc_agg(h2, packed, zeros_nd)
    return _tc_out(p2, nd, b2[None], a2[None])
